# Initial kernel scaffold; baseline (speedup 1.0000x reference)
import jax
import jax.numpy as jnp
from jax.experimental import pallas as pl


def kernel(x, degree, node_table, degree_table, graph_token):
    raise NotImplementedError("write your pallas kernel here")


# speedup vs baseline: 4.9463x; 4.9463x over previous
import functools

import jax
import jax.numpy as jnp
from jax import lax
from jax.experimental import pallas as pl
from jax.experimental.pallas import tpu as pltpu
from jax.experimental.pallas import tpu_sc as plsc

N_GRAPH = 256
N_NODE = 128
N_FEAT = 9
EMB = 128
OUT_ROWS = N_GRAPH * (N_NODE + 1)

NC = 2
NS = 16
NW = NC * NS

GRAPHS_PER_W = N_GRAPH // NW
SLOTS_PER_W = GRAPHS_PER_W * N_NODE
CHUNK = 32
CHUNKS_PER_W = SLOTS_PER_W // CHUNK
CHUNKS_PER_G = N_NODE // CHUNK
IDX_PER_CHUNK = CHUNK * N_FEAT
GATHER_SPLIT = 3
GIDX = IDX_PER_CHUNK // GATHER_SPLIT


def _sc_body(x_hbm, deg_hbm, node_hbm, degt_hbm, tok_hbm, out_hbm,
             idx_v, deg_idx_v, row_idx_v, rows_v, deg_rows_v, out_v,
             tok_rows_v, tok_idx_v, sem):
    cid = lax.axis_index("c")
    sid = lax.axis_index("s")
    wid = sid * NC + cid

    pltpu.sync_copy(tok_hbm, tok_rows_v.at[pl.ds(0, 1)])
    for v in range(EMB // 16):
        sl = pl.ds(v * 16, 16)
        tv = tok_rows_v[0, sl]
        for i in range(1, 16):
            tok_rows_v[i, sl] = tv
    lane = lax.iota(jnp.int32, 16)
    tok_idx_v[pl.ds(0, 16)] = (wid * GRAPHS_PER_W + lane % GRAPHS_PER_W) * (
        N_NODE + 1)
    pltpu.async_copy(tok_rows_v, out_hbm.at[tok_idx_v], sem).wait()

    def chunk_body(c, _):
        p0 = wid * SLOTS_PER_W + c * CHUNK
        g = p0 // N_NODE
        pltpu.sync_copy(x_hbm.at[pl.ds(p0 * N_FEAT, IDX_PER_CHUNK)], idx_v)
        pltpu.sync_copy(deg_hbm.at[pl.ds(p0, CHUNK)], deg_idx_v)
        cps = []
        for i in range(GATHER_SPLIT):
            cps.append(pltpu.async_copy(
                node_hbm.at[idx_v.at[pl.ds(i * GIDX, GIDX)]],
                rows_v.at[pl.ds(i * GIDX, GIDX)], sem))
        cps.append(pltpu.async_copy(degt_hbm.at[deg_idx_v], deg_rows_v, sem))
        for cp in cps:
            cp.wait()

        def slot_body(p, _):
            r0 = p * N_FEAT
            for v in range(EMB // 16):
                sl = pl.ds(v * 16, 16)
                acc = deg_rows_v[p, sl]
                for j in range(N_FEAT):
                    acc = acc + rows_v[r0 + j, sl]
                out_v[p, sl] = acc
            return 0

        lax.fori_loop(0, CHUNK, slot_body, 0)

        row0 = p0 + g + 1
        for v in range(CHUNK // 16):
            row_idx_v[pl.ds(v * 16, 16)] = row0 + v * 16 + lax.iota(
                jnp.int32, 16)
        pltpu.async_copy(out_v, out_hbm.at[row_idx_v], sem).wait()
        return 0

    lax.fori_loop(0, CHUNKS_PER_W, chunk_body, 0)


@jax.jit
def _graph_node_features(x_flat, deg_flat, node_table, degree_table,
                         graph_token):
    mesh = plsc.VectorSubcoreMesh(core_axis_name="c", subcore_axis_name="s")
    out = pl.kernel(
        _sc_body,
        out_type=jax.ShapeDtypeStruct((OUT_ROWS, EMB), jnp.float32),
        mesh=mesh,
        scratch_types=[
            pltpu.VMEM((IDX_PER_CHUNK,), jnp.int32),
            pltpu.VMEM((CHUNK,), jnp.int32),
            pltpu.VMEM((CHUNK,), jnp.int32),
            pltpu.VMEM((IDX_PER_CHUNK, EMB), jnp.float32),
            pltpu.VMEM((CHUNK, EMB), jnp.float32),
            pltpu.VMEM((CHUNK, EMB), jnp.float32),
            pltpu.VMEM((16, EMB), jnp.float32),
            pltpu.VMEM((16,), jnp.int32),
            pltpu.SemaphoreType.DMA,
        ],
    )(x_flat, deg_flat, node_table, degree_table, graph_token)
    return out.reshape(N_GRAPH, N_NODE + 1, EMB)


def kernel(x, degree, node_table, degree_table, graph_token):
    x_flat = x.reshape(-1).astype(jnp.int32)
    deg_flat = degree.reshape(-1).astype(jnp.int32)
    return _graph_node_features(x_flat, deg_flat, node_table, degree_table,
                                graph_token)


# speedup vs baseline: 5.9175x; 1.1964x over previous
import jax
import jax.numpy as jnp
from jax import lax
from jax.experimental import pallas as pl
from jax.experimental.pallas import tpu as pltpu
from jax.experimental.pallas import tpu_sc as plsc

N_GRAPH = 256
N_NODE = 128
N_FEAT = 9
EMB = 128
OUT_ROWS = N_GRAPH * (N_NODE + 1)

NC = 2
NS = 16
NW = NC * NS

GRAPHS_PER_W = N_GRAPH // NW
SLOTS_PER_W = GRAPHS_PER_W * N_NODE
CHUNK = 32
CHUNKS_PER_W = SLOTS_PER_W // CHUNK
IDX_PER_CHUNK = CHUNK * N_FEAT
GATHER_SPLIT = 3
GIDX = IDX_PER_CHUNK // GATHER_SPLIT
NBUF = 2


def _sc_body(x_hbm, deg_hbm, node_hbm, degt_hbm, tok_hbm, out_hbm,
             ix00, ix01, ix02, ix10, ix11, ix12, dg0, dg1, ri0, ri1,
             rows_v, deg_rows_v, out_v,
             tok_rows_v, tok_idx_v, semg0, semg1, semo0, semo1):
    cid = lax.axis_index("c")
    sid = lax.axis_index("s")
    wid = sid * NC + cid
    semg = (semg0, semg1)
    semo = (semo0, semo1)
    node_idx = ((ix00, ix01, ix02), (ix10, ix11, ix12))
    deg_idx = (dg0, dg1)
    row_idx = (ri0, ri1)
    lane = lax.iota(jnp.int32, 16)

    pltpu.sync_copy(tok_hbm, tok_rows_v.at[pl.ds(0, 1)])
    for v in range(EMB // 16):
        sl = pl.ds(v * 16, 16)
        tv = tok_rows_v[0, sl]
        for i in range(1, 16):
            tok_rows_v[i, sl] = tv
    tok_idx_v[pl.ds(0, 16)] = (wid * GRAPHS_PER_W + lane % GRAPHS_PER_W) * (
        N_NODE + 1)
    pltpu.async_copy(tok_rows_v, out_hbm.at[tok_idx_v], semg0).wait()

    def issue(chunk, b):
        p0 = wid * SLOTS_PER_W + chunk * CHUNK
        for i in range(GATHER_SPLIT):
            pltpu.sync_copy(x_hbm.at[pl.ds(p0 * N_FEAT + i * GIDX, GIDX)],
                            node_idx[b][i])
        pltpu.sync_copy(deg_hbm.at[pl.ds(p0, CHUNK)], deg_idx[b])
        for i in range(GATHER_SPLIT):
            pltpu.async_copy(
                node_hbm.at[node_idx[b][i]],
                rows_v.at[b, pl.ds(i * GIDX, GIDX)], semg[b])
        pltpu.async_copy(degt_hbm.at[deg_idx[b]], deg_rows_v.at[b],
                         semg[b])

    def drain_gather(b):
        for i in range(GATHER_SPLIT):
            pltpu.make_async_copy(
                node_hbm.at[node_idx[b][i]],
                rows_v.at[b, pl.ds(i * GIDX, GIDX)], semg[b]).wait()
        pltpu.make_async_copy(degt_hbm.at[deg_idx[b]], deg_rows_v.at[b],
                              semg[b]).wait()

    def drain_scatter(b):
        pltpu.make_async_copy(out_v.at[b], out_hbm.at[row_idx[b]],
                              semo[b]).wait()

    issue(0, 0)
    issue(1, 1)

    def pair_body(i, _):
        for b in range(NBUF):
            chunk = i * NBUF + b
            drain_gather(b)

            @pl.when(chunk >= NBUF)
            def _():
                drain_scatter(b)

            def slot_body(p, _):
                r0 = p * N_FEAT
                for v in range(EMB // 16):
                    sl = pl.ds(v * 16, 16)
                    acc = deg_rows_v[b, p, sl]
                    for j in range(N_FEAT):
                        acc = acc + rows_v[b, r0 + j, sl]
                    out_v[b, p, sl] = acc
                return 0

            lax.fori_loop(0, CHUNK, slot_body, 0)

            p0 = wid * SLOTS_PER_W + chunk * CHUNK
            row0 = p0 + p0 // N_NODE + 1
            for v in range(CHUNK // 16):
                row_idx[b][pl.ds(v * 16, 16)] = row0 + v * 16 + lane
            pltpu.async_copy(out_v.at[b], out_hbm.at[row_idx[b]],
                             semo[b])

            nxt = chunk + NBUF

            @pl.when(nxt < CHUNKS_PER_W)
            def _():
                issue(nxt, b)

        return 0

    lax.fori_loop(0, CHUNKS_PER_W // NBUF, pair_body, 0)
    for b in range(NBUF):
        drain_scatter(b)


@jax.jit
def _graph_node_features(x_flat, deg_flat, node_table, degree_table,
                         graph_token):
    mesh = plsc.VectorSubcoreMesh(core_axis_name="c", subcore_axis_name="s")
    out = pl.kernel(
        _sc_body,
        out_type=jax.ShapeDtypeStruct((OUT_ROWS, EMB), jnp.float32),
        mesh=mesh,
        scratch_types=[
            pltpu.VMEM((GIDX,), jnp.int32),
            pltpu.VMEM((GIDX,), jnp.int32),
            pltpu.VMEM((GIDX,), jnp.int32),
            pltpu.VMEM((GIDX,), jnp.int32),
            pltpu.VMEM((GIDX,), jnp.int32),
            pltpu.VMEM((GIDX,), jnp.int32),
            pltpu.VMEM((CHUNK,), jnp.int32),
            pltpu.VMEM((CHUNK,), jnp.int32),
            pltpu.VMEM((CHUNK,), jnp.int32),
            pltpu.VMEM((CHUNK,), jnp.int32),
            pltpu.VMEM((NBUF, IDX_PER_CHUNK, EMB), jnp.float32),
            pltpu.VMEM((NBUF, CHUNK, EMB), jnp.float32),
            pltpu.VMEM((NBUF, CHUNK, EMB), jnp.float32),
            pltpu.VMEM((16, EMB), jnp.float32),
            pltpu.VMEM((16,), jnp.int32),
            pltpu.SemaphoreType.DMA,
            pltpu.SemaphoreType.DMA,
            pltpu.SemaphoreType.DMA,
            pltpu.SemaphoreType.DMA,
        ],
    )(x_flat, deg_flat, node_table, degree_table, graph_token)
    return out.reshape(N_GRAPH, N_NODE + 1, EMB)


def kernel(x, degree, node_table, degree_table, graph_token):
    x_flat = x.reshape(-1).astype(jnp.int32)
    deg_flat = degree.reshape(-1).astype(jnp.int32)
    return _graph_node_features(x_flat, deg_flat, node_table, degree_table,
                                graph_token)


# speedup vs baseline: 11.6918x; 1.9758x over previous
import jax
import jax.numpy as jnp
from jax import lax
from jax.experimental import pallas as pl
from jax.experimental.pallas import tpu as pltpu
from jax.experimental.pallas import tpu_sc as plsc

N_GRAPH = 256
N_NODE = 128
N_FEAT = 9
EMB = 128
OUT_ROWS = N_GRAPH * (N_NODE + 1)

NC = 2
NS = 16
NW = NC * NS

GPW = N_GRAPH // NW
CHUNK = N_NODE
IDXC = N_FEAT * CHUNK
NBUF = 3


def _sc_body(xt_hbm, deg_hbm, node_hbm, degt_hbm, tok_hbm, out_hbm,
             nix, dgx, rix, acc_v, tok_rows_v, tok_idx_v,
             semi, semd, semg, semo):
    cid = lax.axis_index("c")
    sid = lax.axis_index("s")
    wid = sid * NC + cid
    lane = lax.iota(jnp.int32, 16)

    pltpu.sync_copy(tok_hbm, tok_rows_v.at[pl.ds(0, 1)])
    for v in range(EMB // 16):
        sl = pl.ds(v * 16, 16)
        tv = tok_rows_v[0, sl]
        for i in range(1, 16):
            tok_rows_v[i, sl] = tv
    tok_idx_v[pl.ds(0, 16)] = (wid * GPW + lane % GPW) * (N_NODE + 1)
    pltpu.async_copy(tok_rows_v, out_hbm.at[tok_idx_v], semo[0]).wait()

    def fetch_idx(c):
        b = c % NBUF
        g0 = wid * GPW + c
        pltpu.async_copy(xt_hbm.at[pl.ds(g0 * IDXC, IDXC)], nix[b], semi[b])
        pltpu.async_copy(deg_hbm.at[pl.ds(g0 * CHUNK, CHUNK)], dgx[b],
                         semi[b])

    def drain_idx(c):
        b = c % NBUF
        pltpu.make_async_copy(xt_hbm.at[pl.ds(0, IDXC)], nix[b],
                              semi[b]).wait()
        pltpu.make_async_copy(deg_hbm.at[pl.ds(0, CHUNK)], dgx[b],
                              semi[b]).wait()

    def issue_deg(c):
        b = c % NBUF
        pltpu.async_copy(degt_hbm.at[dgx[b]], acc_v.at[b], semd[b])

    def drain_deg(c):
        b = c % NBUF
        pltpu.make_async_copy(degt_hbm.at[dgx[b]], acc_v.at[b],
                              semd[b]).wait()

    def issue_nodes(c):
        b = c % NBUF
        for j in range(N_FEAT):
            pltpu.async_copy(
                node_hbm.at[nix[b].at[pl.ds(j * CHUNK, CHUNK)]],
                acc_v.at[b], semg[b], add=True)

    def drain_nodes(c):
        b = c % NBUF
        for j in range(N_FEAT):
            pltpu.make_async_copy(
                node_hbm.at[nix[b].at[pl.ds(j * CHUNK, CHUNK)]],
                acc_v.at[b], semg[b]).wait()

    def issue_scatter(c):
        b = c % NBUF
        row0 = (wid * GPW + c) * (N_NODE + 1) + 1
        for v in range(CHUNK // 16):
            rix[b][pl.ds(v * 16, 16)] = row0 + v * 16 + lane
        pltpu.async_copy(acc_v.at[b], out_hbm.at[rix[b]], semo[b])

    def drain_scatter(c):
        b = c % NBUF
        pltpu.make_async_copy(acc_v.at[b], out_hbm.at[rix[b]],
                              semo[b]).wait()

    fetch_idx(0)
    fetch_idx(1)
    drain_idx(0)
    issue_deg(0)
    drain_deg(0)
    issue_nodes(0)

    for c in range(GPW):
        if c >= 1:
            drain_scatter(c - 1)
        if c + 2 < GPW:
            fetch_idx(c + 2)
        if c + 1 < GPW:
            drain_idx(c + 1)
            issue_deg(c + 1)
        drain_nodes(c)
        issue_scatter(c)
        if c + 1 < GPW:
            drain_deg(c + 1)
            issue_nodes(c + 1)
    drain_scatter(GPW - 1)


@jax.jit
def _graph_node_features(xt_flat, deg_flat, node_table, degree_table,
                         graph_token):
    mesh = plsc.VectorSubcoreMesh(core_axis_name="c", subcore_axis_name="s")
    out = pl.kernel(
        _sc_body,
        out_type=jax.ShapeDtypeStruct((OUT_ROWS, EMB), jnp.float32),
        mesh=mesh,
        scratch_types=[
            [pltpu.VMEM((IDXC,), jnp.int32) for _ in range(NBUF)],
            [pltpu.VMEM((CHUNK,), jnp.int32) for _ in range(NBUF)],
            [pltpu.VMEM((CHUNK,), jnp.int32) for _ in range(NBUF)],
            pltpu.VMEM((NBUF, CHUNK, EMB), jnp.float32),
            pltpu.VMEM((16, EMB), jnp.float32),
            pltpu.VMEM((16,), jnp.int32),
            [pltpu.SemaphoreType.DMA for _ in range(NBUF)],
            [pltpu.SemaphoreType.DMA for _ in range(NBUF)],
            [pltpu.SemaphoreType.DMA for _ in range(NBUF)],
            [pltpu.SemaphoreType.DMA for _ in range(NBUF)],
        ],
    )(xt_flat, deg_flat, node_table, degree_table, graph_token)
    return out.reshape(N_GRAPH, N_NODE + 1, EMB)


def kernel(x, degree, node_table, degree_table, graph_token):
    xt_flat = x.astype(jnp.int32).transpose(0, 2, 1).reshape(-1)
    deg_flat = degree.reshape(-1).astype(jnp.int32)
    return _graph_node_features(xt_flat, deg_flat, node_table, degree_table,
                                graph_token)


# speedup vs baseline: 12.4808x; 1.0675x over previous
import jax
import jax.numpy as jnp
from jax import lax
from jax.experimental import pallas as pl
from jax.experimental.pallas import tpu as pltpu
from jax.experimental.pallas import tpu_sc as plsc

N_GRAPH = 256
N_NODE = 128
N_FEAT = 9
EMB = 128
OUT_ROWS = N_GRAPH * (N_NODE + 1)

NC = 2
NS = 16
NW = NC * NS

GPW = N_GRAPH // NW
CHUNK = N_NODE
IDXC = N_FEAT * CHUNK
NBUF = 3


def _sc_body(xt_hbm, deg_hbm, node_hbm, degt_hbm, tok_hbm, out_hbm,
             nix, dgx, rix, acc_v, degt_sh, tok_rows_v, tok_idx_v,
             semi, semd, semg, semo):
    cid = lax.axis_index("c")
    sid = lax.axis_index("s")
    wid = sid * NC + cid
    lane = lax.iota(jnp.int32, 16)

    @pl.when(sid == 0)
    def _():
        pltpu.sync_copy(degt_hbm, degt_sh)
    plsc.subcore_barrier()

    pltpu.sync_copy(tok_hbm, tok_rows_v.at[pl.ds(0, 1)])
    for v in range(EMB // 16):
        sl = pl.ds(v * 16, 16)
        tv = tok_rows_v[0, sl]
        for i in range(1, 16):
            tok_rows_v[i, sl] = tv
    tok_idx_v[pl.ds(0, 16)] = (wid * GPW + lane % GPW) * (N_NODE + 1)
    pltpu.async_copy(tok_rows_v, out_hbm.at[tok_idx_v], semo[0]).wait()

    def fetch_idx(c):
        b = c % NBUF
        g0 = wid * GPW + c
        pltpu.async_copy(xt_hbm.at[pl.ds(g0 * IDXC, IDXC)], nix[b], semi[b])
        pltpu.async_copy(deg_hbm.at[pl.ds(g0 * CHUNK, CHUNK)], dgx[b],
                         semi[b])

    def drain_idx(c):
        b = c % NBUF
        pltpu.make_async_copy(xt_hbm.at[pl.ds(0, IDXC)], nix[b],
                              semi[b]).wait()
        pltpu.make_async_copy(deg_hbm.at[pl.ds(0, CHUNK)], dgx[b],
                              semi[b]).wait()

    def issue_deg(c):
        b = c % NBUF
        pltpu.async_copy(degt_sh.at[dgx[b]], acc_v.at[b], semd[b])

    def drain_deg(c):
        b = c % NBUF
        pltpu.make_async_copy(degt_sh.at[dgx[b]], acc_v.at[b],
                              semd[b]).wait()

    def issue_nodes(c):
        b = c % NBUF
        for j in range(N_FEAT):
            pltpu.async_copy(
                node_hbm.at[nix[b].at[pl.ds(j * CHUNK, CHUNK)]],
                acc_v.at[b], semg[b], add=True)

    def drain_nodes(c):
        b = c % NBUF
        for j in range(N_FEAT):
            pltpu.make_async_copy(
                node_hbm.at[nix[b].at[pl.ds(j * CHUNK, CHUNK)]],
                acc_v.at[b], semg[b]).wait()

    def issue_scatter(c):
        b = c % NBUF
        row0 = (wid * GPW + c) * (N_NODE + 1) + 1
        for v in range(CHUNK // 16):
            rix[b][pl.ds(v * 16, 16)] = row0 + v * 16 + lane
        pltpu.async_copy(acc_v.at[b], out_hbm.at[rix[b]], semo[b])

    def drain_scatter(c):
        b = c % NBUF
        pltpu.make_async_copy(acc_v.at[b], out_hbm.at[rix[b]],
                              semo[b]).wait()

    fetch_idx(0)
    fetch_idx(1)
    drain_idx(0)
    issue_deg(0)
    drain_deg(0)
    issue_nodes(0)

    for c in range(GPW):
        if c >= 1:
            drain_scatter(c - 1)
        if c + 2 < GPW:
            fetch_idx(c + 2)
        if c + 1 < GPW:
            drain_idx(c + 1)
            issue_deg(c + 1)
        drain_nodes(c)
        issue_scatter(c)
        if c + 1 < GPW:
            drain_deg(c + 1)
            issue_nodes(c + 1)
    drain_scatter(GPW - 1)


@jax.jit
def _graph_node_features(xt_flat, deg_flat, node_table, degree_table,
                         graph_token):
    mesh = plsc.VectorSubcoreMesh(core_axis_name="c", subcore_axis_name="s")
    out = pl.kernel(
        _sc_body,
        out_type=jax.ShapeDtypeStruct((OUT_ROWS, EMB), jnp.float32),
        mesh=mesh,
        scratch_types=[
            [pltpu.VMEM((IDXC,), jnp.int32) for _ in range(NBUF)],
            [pltpu.VMEM((CHUNK,), jnp.int32) for _ in range(NBUF)],
            [pltpu.VMEM((CHUNK,), jnp.int32) for _ in range(NBUF)],
            pltpu.VMEM((NBUF, CHUNK, EMB), jnp.float32),
            pltpu.VMEM_SHARED((512, EMB), jnp.float32),
            pltpu.VMEM((16, EMB), jnp.float32),
            pltpu.VMEM((16,), jnp.int32),
            [pltpu.SemaphoreType.DMA for _ in range(NBUF)],
            [pltpu.SemaphoreType.DMA for _ in range(NBUF)],
            [pltpu.SemaphoreType.DMA for _ in range(NBUF)],
            [pltpu.SemaphoreType.DMA for _ in range(NBUF)],
        ],
    )(xt_flat, deg_flat, node_table, degree_table, graph_token)
    return out.reshape(N_GRAPH, N_NODE + 1, EMB)


def kernel(x, degree, node_table, degree_table, graph_token):
    xt_flat = x.astype(jnp.int32).transpose(0, 2, 1).reshape(-1)
    deg_flat = degree.reshape(-1).astype(jnp.int32)
    return _graph_node_features(xt_flat, deg_flat, node_table, degree_table,
                                graph_token)
